# Initial kernel scaffold; baseline (speedup 1.0000x reference)
import jax
import jax.numpy as jnp
from jax.experimental import pallas as pl


def kernel(input_spikes, weight, labels):
    raise NotImplementedError("write your pallas kernel here")


# speedup vs baseline: 1.6871x; 1.6871x over previous
import functools

import jax
import jax.numpy as jnp
from jax import lax
from jax.experimental import pallas as pl
from jax.experimental.pallas import tpu as pltpu
import numpy as np

B, CIN, S, T = 32, 1, 128, 64
O, N = 1, 2048
STEP, LEAK = 16, 32
KS = STEP + LEAK
PAD = int(np.ceil((KS + STEP) / 2))
FODEP = KS
THETA = 0.1 * (S * CIN)
TOUT = T + 2 * PAD - KS + 1
ROWS = B * TOUT
KDIM = KS * S
KAUG = KDIM + B
KPAD = 6272

R_TILE, N_TILE, K_TILE = 288, 512, 896
R_BLKS, N_BLKS, K_BLKS = ROWS // R_TILE, N // N_TILE, KPAD // K_TILE


def _builder_body(wt_ref, lab_ref, out_ref):
    n_blk = pl.program_id(0)
    w = wt_ref[...]
    for k in range(KS):
        kk = float(KS - 1 - k)
        val = jnp.maximum(0.0, jnp.minimum(kk / STEP, 1.5 * w - kk / LEAK))
        out_ref[k * S:(k + 1) * S, :] = val
    lab = lab_ref[...]
    n_iota = lax.broadcasted_iota(jnp.int32, (B, N_TILE), 1) + n_blk * N_TILE
    out_ref[KDIM:KAUG, :] = jnp.where(lab == n_iota, 6.0, 0.0)
    out_ref[KAUG:KPAD, :] = jnp.zeros((KPAD - KAUG, N_TILE), jnp.float32)


def _build_w2(weight, labels):
    wt = weight.T
    lab = labels.reshape(B, 1)
    return pl.pallas_call(
        _builder_body,
        grid=(N_BLKS,),
        in_specs=[
            pl.BlockSpec((S, N_TILE), lambda n: (0, n)),
            pl.BlockSpec((B, 1), lambda n: (0, 0)),
        ],
        out_specs=pl.BlockSpec((KPAD, N_TILE), lambda n: (0, n)),
        out_shape=jax.ShapeDtypeStruct((KPAD, N), jnp.float32),
    )(wt, lab)


def _matmul_body(a_ref, w_ref, m_ref, i_ref, acc_ref, best_ref, besti_ref):
    n_blk = pl.program_id(1)
    k_blk = pl.program_id(2)
    dot = jnp.dot(a_ref[...], w_ref[...], preferred_element_type=jnp.float32)

    @pl.when(k_blk == 0)
    def _():
        acc_ref[...] = dot

    @pl.when(k_blk > 0)
    def _():
        acc_ref[...] += dot

    @pl.when(k_blk == K_BLKS - 1)
    def _():
        tile = acc_ref[...]
        tmax = jnp.max(tile, axis=1, keepdims=True)
        targ = (jnp.argmax(tile, axis=1).astype(jnp.int32)
                + n_blk * N_TILE)[:, None]

        @pl.when(n_blk == 0)
        def _():
            best_ref[...] = tmax
            besti_ref[...] = targ

        @pl.when(n_blk > 0)
        def _():
            upd = tmax > best_ref[...]
            best_ref[...] = jnp.where(upd, tmax, best_ref[...])
            besti_ref[...] = jnp.where(upd, targ, besti_ref[...])

        @pl.when(n_blk == N_BLKS - 1)
        def _():
            m_ref[...] = best_ref[...]
            i_ref[...] = besti_ref[...]


def _max_argmax(a_aug, w2):
    return pl.pallas_call(
        _matmul_body,
        grid=(R_BLKS, N_BLKS, K_BLKS),
        in_specs=[
            pl.BlockSpec((R_TILE, K_TILE), lambda r, n, k: (r, k)),
            pl.BlockSpec((K_TILE, N_TILE), lambda r, n, k: (k, n)),
        ],
        out_specs=[
            pl.BlockSpec((R_TILE, 1), lambda r, n, k: (r, 0)),
            pl.BlockSpec((R_TILE, 1), lambda r, n, k: (r, 0)),
        ],
        out_shape=[
            jax.ShapeDtypeStruct((ROWS, 1), jnp.float32),
            jax.ShapeDtypeStruct((ROWS, 1), jnp.int32),
        ],
        scratch_shapes=[
            pltpu.VMEM((R_TILE, N_TILE), jnp.float32),
            pltpu.VMEM((R_TILE, 1), jnp.float32),
            pltpu.VMEM((R_TILE, 1), jnp.int32),
        ],
    )(a_aug, w2)


def _onehot_body(m_ref, a_ref, out_ref):
    mv = m_ref[...].reshape(1, TOUT)
    a_row = a_ref[...].reshape(1, TOUT)
    q = mv > jnp.float32(THETA)
    tio = lax.broadcasted_iota(jnp.int32, (1, TOUT), 1)
    big = jnp.int32(4 * TOUT)
    first = jnp.min(jnp.where(q, tio, big), axis=1, keepdims=True)
    q2 = jnp.logical_and(q, tio >= first + FODEP)
    second = jnp.min(jnp.where(q2, tio, big), axis=1, keepdims=True)
    sp = jnp.logical_or(tio == first, tio == second)
    n_iota = lax.broadcasted_iota(jnp.int32, (N, TOUT), 0)
    hit = jnp.logical_and(n_iota == a_row, sp)
    out_ref[0, 0] = jnp.where(hit, 1.0, 0.0).astype(jnp.float32)


def _onehot(m, aidx):
    return pl.pallas_call(
        _onehot_body,
        grid=(B,),
        in_specs=[
            pl.BlockSpec((1, 1, TOUT), lambda b: (b, 0, 0)),
            pl.BlockSpec((1, 1, TOUT), lambda b: (b, 0, 0)),
        ],
        out_specs=pl.BlockSpec((1, 1, N, TOUT), lambda b: (b, 0, 0, 0)),
        out_shape=jax.ShapeDtypeStruct((B, O, N, TOUT), jnp.float32),
    )(m.reshape(B, 1, TOUT), aidx.reshape(B, 1, TOUT))


def _build_a(input_spikes):
    x = input_spikes.reshape(B, S, T)
    xp = jnp.pad(x, ((0, 0), (0, 0), (PAD, PAD)))
    cols = [xp[:, :, k:k + TOUT] for k in range(KS)]
    a4 = jnp.stack(cols, axis=0)
    a = a4.transpose(1, 3, 0, 2).reshape(ROWS, KDIM)
    b_onehot = jnp.repeat(jnp.eye(B, dtype=jnp.float32), TOUT, axis=0)
    zpad = jnp.zeros((ROWS, KPAD - KAUG), jnp.float32)
    return jnp.concatenate([a, b_onehot, zpad], axis=1)


@jax.jit
def kernel(input_spikes, weight, labels):
    a_aug = _build_a(input_spikes)
    w2 = _build_w2(weight, labels.astype(jnp.int32))
    m, aidx = _max_argmax(a_aug, w2)
    return _onehot(m, aidx)


# speedup vs baseline: 2.4934x; 1.4779x over previous
import functools

import jax
import jax.numpy as jnp
from jax import lax
from jax.experimental import pallas as pl
from jax.experimental.pallas import tpu as pltpu
import numpy as np

B, CIN, S, T = 32, 1, 128, 64
O, N = 1, 2048
STEP, LEAK = 16, 32
KS = STEP + LEAK
PAD = int(np.ceil((KS + STEP) / 2))
FODEP = KS
THETA = 0.1 * (S * CIN)
TOUT = T + 2 * PAD - KS + 1
ROWS = B * TOUT
KDIM = KS * S
KAUG = KDIM + B
KPAD = 6272

R_TILE, N_TILE, K_TILE = 2592, 1024, 896
R_BLKS, N_BLKS, K_BLKS = ROWS // R_TILE, N // N_TILE, KPAD // K_TILE
BN_TILE = 512
BN_BLKS = N // BN_TILE


def _builder_body(wt_ref, lab_ref, out_ref):
    n_blk = pl.program_id(0)
    w = wt_ref[...]
    for k in range(KS):
        kk = float(KS - 1 - k)
        val = jnp.maximum(0.0, jnp.minimum(kk / STEP, 1.5 * w - kk / LEAK))
        out_ref[k * S:(k + 1) * S, :] = val
    lab = lab_ref[...]
    n_iota = lax.broadcasted_iota(jnp.int32, (B, BN_TILE), 1) + n_blk * BN_TILE
    out_ref[KDIM:KAUG, :] = jnp.where(lab == n_iota, 6.0, 0.0)
    out_ref[KAUG:KPAD, :] = jnp.zeros((KPAD - KAUG, BN_TILE), jnp.float32)


def _build_w2(weight, labels):
    wt = weight.T
    lab = labels.reshape(B, 1)
    return pl.pallas_call(
        _builder_body,
        grid=(BN_BLKS,),
        in_specs=[
            pl.BlockSpec((S, BN_TILE), lambda n: (0, n)),
            pl.BlockSpec((B, 1), lambda n: (0, 0)),
        ],
        out_specs=pl.BlockSpec((KPAD, BN_TILE), lambda n: (0, n)),
        out_shape=jax.ShapeDtypeStruct((KPAD, N), jnp.float32),
    )(wt, lab)


def _matmul_body(a_ref, w_ref, m_ref, i_ref, acc_ref, best_ref, besti_ref):
    n_blk = pl.program_id(1)
    k_blk = pl.program_id(2)
    dot = jnp.dot(a_ref[...], w_ref[...], preferred_element_type=jnp.float32)

    @pl.when(k_blk == 0)
    def _():
        acc_ref[...] = dot

    @pl.when(k_blk > 0)
    def _():
        acc_ref[...] += dot

    @pl.when(k_blk == K_BLKS - 1)
    def _():
        tile = acc_ref[...]
        tmax = jnp.max(tile, axis=1, keepdims=True)
        targ = (jnp.argmax(tile, axis=1).astype(jnp.int32)
                + n_blk * N_TILE)[:, None]

        @pl.when(n_blk == 0)
        def _():
            best_ref[...] = tmax
            besti_ref[...] = targ

        @pl.when(n_blk > 0)
        def _():
            upd = tmax > best_ref[...]
            best_ref[...] = jnp.where(upd, tmax, best_ref[...])
            besti_ref[...] = jnp.where(upd, targ, besti_ref[...])

        @pl.when(n_blk == N_BLKS - 1)
        def _():
            m_ref[...] = best_ref[...]
            i_ref[...] = besti_ref[...]


def _max_argmax(a_aug, w2):
    return pl.pallas_call(
        _matmul_body,
        grid=(R_BLKS, N_BLKS, K_BLKS),
        in_specs=[
            pl.BlockSpec((R_TILE, K_TILE), lambda r, n, k: (r, k)),
            pl.BlockSpec((K_TILE, N_TILE), lambda r, n, k: (k, n)),
        ],
        out_specs=[
            pl.BlockSpec((R_TILE, 1), lambda r, n, k: (r, 0)),
            pl.BlockSpec((R_TILE, 1), lambda r, n, k: (r, 0)),
        ],
        out_shape=[
            jax.ShapeDtypeStruct((ROWS, 1), jnp.float32),
            jax.ShapeDtypeStruct((ROWS, 1), jnp.int32),
        ],
        scratch_shapes=[
            pltpu.VMEM((R_TILE, N_TILE), jnp.float32),
            pltpu.VMEM((R_TILE, 1), jnp.float32),
            pltpu.VMEM((R_TILE, 1), jnp.int32),
        ],
    )(a_aug, w2)


def _onehot_body(m_ref, a_ref, out_ref):
    mv = m_ref[...].reshape(1, TOUT)
    a_row = a_ref[...].reshape(1, TOUT)
    q = mv > jnp.float32(THETA)
    tio = lax.broadcasted_iota(jnp.int32, (1, TOUT), 1)
    big = jnp.int32(4 * TOUT)
    first = jnp.min(jnp.where(q, tio, big), axis=1, keepdims=True)
    q2 = jnp.logical_and(q, tio >= first + FODEP)
    second = jnp.min(jnp.where(q2, tio, big), axis=1, keepdims=True)
    sp = jnp.logical_or(tio == first, tio == second)
    n_iota = lax.broadcasted_iota(jnp.int32, (N, TOUT), 0)
    hit = jnp.logical_and(n_iota == a_row, sp)
    out_ref[0, 0] = jnp.where(hit, 1.0, 0.0).astype(jnp.float32)


def _onehot(m, aidx):
    return pl.pallas_call(
        _onehot_body,
        grid=(B,),
        in_specs=[
            pl.BlockSpec((1, 1, TOUT), lambda b: (b, 0, 0)),
            pl.BlockSpec((1, 1, TOUT), lambda b: (b, 0, 0)),
        ],
        out_specs=pl.BlockSpec((1, 1, N, TOUT), lambda b: (b, 0, 0, 0)),
        out_shape=jax.ShapeDtypeStruct((B, O, N, TOUT), jnp.float32),
    )(m.reshape(B, 1, TOUT), aidx.reshape(B, 1, TOUT))


def _build_a(input_spikes):
    x = input_spikes.reshape(B, S, T)
    xp = jnp.pad(x, ((0, 0), (0, 0), (PAD, PAD)))
    cols = [xp[:, :, k:k + TOUT] for k in range(KS)]
    a4 = jnp.stack(cols, axis=0)
    a = a4.transpose(1, 3, 0, 2).reshape(ROWS, KDIM)
    b_onehot = jnp.repeat(jnp.eye(B, dtype=jnp.float32), TOUT, axis=0)
    zpad = jnp.zeros((ROWS, KPAD - KAUG), jnp.float32)
    return jnp.concatenate([a, b_onehot, zpad], axis=1)


@jax.jit
def kernel(input_spikes, weight, labels):
    a_aug = _build_a(input_spikes)
    w2 = _build_w2(weight, labels.astype(jnp.int32))
    m, aidx = _max_argmax(a_aug, w2)
    return _onehot(m, aidx)


# speedup vs baseline: 3.5935x; 1.4412x over previous
import jax
import jax.numpy as jnp
from jax import lax
from jax.experimental import pallas as pl
from jax.experimental.pallas import tpu as pltpu
import numpy as np

B, CIN, S, T = 32, 1, 128, 64
O, N = 1, 2048
STEP, LEAK = 16, 32
KS = STEP + LEAK
PAD = int(np.ceil((KS + STEP) / 2))
FODEP = KS
THETA = 0.1 * (S * CIN)
TOUT = T + 2 * PAD - KS + 1
TP = T + 2 * PAD
MROW = 88
NSH = 8
NQ = KS // NSH

N_TILE = 512
N_BLKS = N // N_TILE


def _builder_body(wt_ref, lab_ref, w3_ref, sup_ref):
    n_blk = pl.program_id(0)
    w = wt_ref[...]
    for k in range(KS):
        kk = float(KS - 1 - k)
        w3_ref[k] = jnp.maximum(0.0, jnp.minimum(kk / STEP, 1.5 * w - kk / LEAK))
    lab = lab_ref[...]
    n_iota = lax.broadcasted_iota(jnp.int32, (B, N_TILE), 1) + n_blk * N_TILE
    sup_ref[:, 0, :] = jnp.where(lab == n_iota, 6.0, 0.0)


def _build_w3(weight, labels):
    wt = weight.T
    lab = labels.reshape(B, 1)
    return pl.pallas_call(
        _builder_body,
        grid=(N_BLKS,),
        in_specs=[
            pl.BlockSpec((S, N_TILE), lambda n: (0, n)),
            pl.BlockSpec((B, 1), lambda n: (0, 0)),
        ],
        out_specs=[
            pl.BlockSpec((KS, S, N_TILE), lambda n: (0, 0, n)),
            pl.BlockSpec((B, 1, N_TILE), lambda n: (0, 0, n)),
        ],
        out_shape=[
            jax.ShapeDtypeStruct((KS, S, N), jnp.float32),
            jax.ShapeDtypeStruct((B, 1, N), jnp.float32),
        ],
    )(wt, lab)


def _conv_body(xss_ref, w_ref, sup_ref, m_ref, a_ref):
    n_blk = pl.program_id(0)
    acc = jnp.zeros((MROW, N_TILE), jnp.float32)
    for q in range(NQ):
        for j in range(NSH):
            k = NSH * q + j
            acc = acc + jnp.dot(
                xss_ref[j, 0, NSH * q:NSH * q + MROW, :], w_ref[k],
                preferred_element_type=jnp.float32)
    acc = acc + sup_ref[...].reshape(1, N_TILE)
    m_ref[0, 0] = jnp.max(acc, axis=1, keepdims=True)
    a_ref[0, 0] = (jnp.argmax(acc, axis=1).astype(jnp.int32)
                   + n_blk * N_TILE)[:, None]


def _conv_max_argmax(xss, w3, sup):
    return pl.pallas_call(
        _conv_body,
        grid=(N_BLKS, B),
        in_specs=[
            pl.BlockSpec((NSH, 1, TP, S), lambda n, b: (0, b, 0, 0)),
            pl.BlockSpec((KS, S, N_TILE), lambda n, b: (0, 0, n)),
            pl.BlockSpec((1, 1, N_TILE), lambda n, b: (b, 0, n)),
        ],
        out_specs=[
            pl.BlockSpec((1, 1, MROW, 1), lambda n, b: (n, b, 0, 0)),
            pl.BlockSpec((1, 1, MROW, 1), lambda n, b: (n, b, 0, 0)),
        ],
        out_shape=[
            jax.ShapeDtypeStruct((N_BLKS, B, MROW, 1), jnp.float32),
            jax.ShapeDtypeStruct((N_BLKS, B, MROW, 1), jnp.int32),
        ],
    )(xss, w3, sup)


def _onehot_body(m_ref, a_ref, out_ref):
    best = m_ref[0, 0]
    bi = a_ref[0, 0]
    for p in range(1, N_BLKS):
        mk = m_ref[p, 0]
        ak = a_ref[p, 0]
        upd = mk > best
        best = jnp.where(upd, mk, best)
        bi = jnp.where(upd, ak, bi)
    tio = lax.broadcasted_iota(jnp.int32, (MROW, 1), 0)
    valid = tio < TOUT
    q = jnp.logical_and(valid, best > jnp.float32(THETA))
    big = jnp.int32(4 * MROW)
    t1 = jnp.min(jnp.where(q, tio, big))
    q2 = jnp.logical_and(q, tio >= t1 + FODEP)
    t2 = jnp.min(jnp.where(q2, tio, big))
    a1 = jnp.sum(jnp.where(tio == t1, bi, 0))
    a2 = jnp.sum(jnp.where(tio == t2, bi, 0))
    n_io = lax.broadcasted_iota(jnp.int32, (N, TOUT), 0)
    t_io = lax.broadcasted_iota(jnp.int32, (N, TOUT), 1)
    hit1 = jnp.logical_and(n_io == a1, t_io == t1)
    hit2 = jnp.logical_and(n_io == a2, t_io == t2)
    out_ref[0, 0] = jnp.where(jnp.logical_or(hit1, hit2), 1.0, 0.0)


def _onehot(mpart, apart):
    return pl.pallas_call(
        _onehot_body,
        grid=(B,),
        in_specs=[
            pl.BlockSpec((N_BLKS, 1, MROW, 1), lambda b: (0, b, 0, 0)),
            pl.BlockSpec((N_BLKS, 1, MROW, 1), lambda b: (0, b, 0, 0)),
        ],
        out_specs=pl.BlockSpec((1, 1, N, TOUT), lambda b: (b, 0, 0, 0)),
        out_shape=jax.ShapeDtypeStruct((B, O, N, TOUT), jnp.float32),
    )(mpart, apart)


def _build_xss(input_spikes):
    x = input_spikes.reshape(B, S, T)
    xp = jnp.pad(x, ((0, 0), (0, 0), (PAD, PAD)))
    xt = jnp.transpose(xp, (0, 2, 1))
    xt = jnp.pad(xt, ((0, 0), (0, NSH), (0, 0)))
    return jnp.stack([xt[:, j:j + TP, :] for j in range(NSH)], axis=0)


@jax.jit
def kernel(input_spikes, weight, labels):
    xss = _build_xss(input_spikes)
    w3, sup = _build_w3(weight, labels.astype(jnp.int32))
    mpart, apart = _conv_max_argmax(xss, w3, sup)
    return _onehot(mpart, apart)


# speedup vs baseline: 3.7704x; 1.0492x over previous
import jax
import jax.numpy as jnp
from jax import lax
from jax.experimental import pallas as pl
from jax.experimental.pallas import tpu as pltpu
import numpy as np

B, CIN, S, T = 32, 1, 128, 64
O, N = 1, 2048
STEP, LEAK = 16, 32
KS = STEP + LEAK
PAD = int(np.ceil((KS + STEP) / 2))
FODEP = KS
THETA = 0.1 * (S * CIN)
TOUT = T + 2 * PAD - KS + 1
TP = T + 2 * PAD
MROW = 88
NSH = 8
NQ = KS // NSH

N_TILE = 512
N_BLKS = N // N_TILE


def _builder_body(wt_ref, lab_ref, w3_ref, sup_ref):
    n_blk = pl.program_id(0)
    w = wt_ref[...]
    for q in range(NQ):
        for j in range(NSH):
            kk = float(KS - 1 - (NSH * q + j))
            w3_ref[q, j * S:(j + 1) * S, :] = jnp.maximum(
                0.0, jnp.minimum(kk / STEP, 1.5 * w - kk / LEAK))
    lab = lab_ref[...]
    n_iota = lax.broadcasted_iota(jnp.int32, (B, N_TILE), 1) + n_blk * N_TILE
    sup_ref[:, 0, :] = jnp.where(lab == n_iota, 6.0, 0.0)


def _build_w3(weight, labels):
    wt = weight.T
    lab = labels.reshape(B, 1)
    return pl.pallas_call(
        _builder_body,
        grid=(N_BLKS,),
        in_specs=[
            pl.BlockSpec((S, N_TILE), lambda n: (0, n)),
            pl.BlockSpec((B, 1), lambda n: (0, 0)),
        ],
        out_specs=[
            pl.BlockSpec((NQ, NSH * S, N_TILE), lambda n: (0, 0, n)),
            pl.BlockSpec((B, 1, N_TILE), lambda n: (0, 0, n)),
        ],
        out_shape=[
            jax.ShapeDtypeStruct((NQ, NSH * S, N), jnp.float32),
            jax.ShapeDtypeStruct((B, 1, N), jnp.float32),
        ],
    )(wt, lab)


def _conv_body(xc_ref, w_ref, sup_ref, m_ref, a_ref):
    n_blk = pl.program_id(0)
    acc = jnp.zeros((MROW, N_TILE), jnp.float32)
    for q in range(NQ):
        acc = acc + jnp.dot(
            xc_ref[0, NSH * q:NSH * q + MROW, :], w_ref[q],
            preferred_element_type=jnp.float32)
    acc = acc + sup_ref[...].reshape(1, N_TILE)
    m_ref[0, 0] = jnp.max(acc, axis=1, keepdims=True)
    a_ref[0, 0] = (jnp.argmax(acc, axis=1).astype(jnp.int32)
                   + n_blk * N_TILE)[:, None]


def _conv_max_argmax(xc, w3, sup):
    return pl.pallas_call(
        _conv_body,
        grid=(N_BLKS, B),
        in_specs=[
            pl.BlockSpec((1, TP, NSH * S), lambda n, b: (b, 0, 0)),
            pl.BlockSpec((NQ, NSH * S, N_TILE), lambda n, b: (0, 0, n)),
            pl.BlockSpec((1, 1, N_TILE), lambda n, b: (b, 0, n)),
        ],
        out_specs=[
            pl.BlockSpec((1, 1, MROW, 1), lambda n, b: (n, b, 0, 0)),
            pl.BlockSpec((1, 1, MROW, 1), lambda n, b: (n, b, 0, 0)),
        ],
        out_shape=[
            jax.ShapeDtypeStruct((N_BLKS, B, MROW, 1), jnp.float32),
            jax.ShapeDtypeStruct((N_BLKS, B, MROW, 1), jnp.int32),
        ],
    )(xc, w3, sup)


def _onehot_body(m_ref, a_ref, out_ref):
    best = m_ref[0, 0]
    bi = a_ref[0, 0]
    for p in range(1, N_BLKS):
        mk = m_ref[p, 0]
        ak = a_ref[p, 0]
        upd = mk > best
        best = jnp.where(upd, mk, best)
        bi = jnp.where(upd, ak, bi)
    tio = lax.broadcasted_iota(jnp.int32, (MROW, 1), 0)
    valid = tio < TOUT
    q = jnp.logical_and(valid, best > jnp.float32(THETA))
    big = jnp.int32(4 * MROW)
    t1 = jnp.min(jnp.where(q, tio, big))
    q2 = jnp.logical_and(q, tio >= t1 + FODEP)
    t2 = jnp.min(jnp.where(q2, tio, big))
    a1 = jnp.sum(jnp.where(tio == t1, bi, 0))
    a2 = jnp.sum(jnp.where(tio == t2, bi, 0))
    n_io = lax.broadcasted_iota(jnp.int32, (N, TOUT), 0)
    t_io = lax.broadcasted_iota(jnp.int32, (N, TOUT), 1)
    hit1 = jnp.logical_and(n_io == a1, t_io == t1)
    hit2 = jnp.logical_and(n_io == a2, t_io == t2)
    out_ref[0, 0] = jnp.where(jnp.logical_or(hit1, hit2), 1.0, 0.0)


def _onehot(mpart, apart):
    return pl.pallas_call(
        _onehot_body,
        grid=(B,),
        in_specs=[
            pl.BlockSpec((N_BLKS, 1, MROW, 1), lambda b: (0, b, 0, 0)),
            pl.BlockSpec((N_BLKS, 1, MROW, 1), lambda b: (0, b, 0, 0)),
        ],
        out_specs=pl.BlockSpec((1, 1, N, TOUT), lambda b: (b, 0, 0, 0)),
        out_shape=jax.ShapeDtypeStruct((B, O, N, TOUT), jnp.float32),
    )(mpart, apart)


def _build_xc(input_spikes):
    x = input_spikes.reshape(B, S, T)
    xp = jnp.pad(x, ((0, 0), (0, 0), (PAD, PAD)))
    xt = jnp.transpose(xp, (0, 2, 1))
    xt = jnp.pad(xt, ((0, 0), (0, NSH), (0, 0)))
    xc = jnp.stack([xt[:, j:j + TP, :] for j in range(NSH)], axis=2)
    return xc.reshape(B, TP, NSH * S)


@jax.jit
def kernel(input_spikes, weight, labels):
    xc = _build_xc(input_spikes)
    w3, sup = _build_w3(weight, labels.astype(jnp.int32))
    mpart, apart = _conv_max_argmax(xc, w3, sup)
    return _onehot(mpart, apart)
